# baseline (speedup 1.0000x reference)
import jax
import jax.numpy as jnp
from jax import lax
from jax.experimental import pallas as pl
from jax.experimental.pallas import tpu as pltpu
from jax.experimental.pallas import tpu_sc as plsc

E = 8
B = 8192
D_IN = 2048
D_OUT = 2048

BM = 256
NB = B // BM + E
NPAD = NB * BM

try:
    _info = plsc.get_sparse_core_info()
    NC, NS = int(_info.num_cores), int(_info.num_subcores)
except Exception:
    NC, NS = 2, 16
NW = NC * NS
RPW = NPAD // NW
TPW = B // NW
CH = 16
NCH1 = RPW // CH
NCH3 = TPW // CH

_MESH = dict(mesh=plsc.VectorSubcoreMesh(core_axis_name="c", subcore_axis_name="s"))


def _pipelined(nch, start_load, start_store):
    lh, sh = {}, {}
    lh[0] = start_load(0, 0)
    for c in range(nch):
        k = c % 2
        lh[c].wait()
        if c + 1 < nch:
            if c - 1 >= 0:
                sh[c - 1].wait()
            lh[c + 1] = start_load(c + 1, 1 - k)
        sh[c] = start_store(c, k)
    sh[nch - 1].wait()
    if nch >= 2:
        sh[nch - 2].wait()


def _gather_body(x_hbm, src_hbm, out_hbm, idx_v, buf0, buf1, gsem, ssem):
    wid = lax.axis_index("s") * NC + lax.axis_index("c")
    base = wid * RPW
    bufs = (buf0, buf1)
    pltpu.sync_copy(src_hbm.at[pl.ds(base, RPW)], idx_v)

    def ld(c, k):
        return pltpu.async_copy(
            x_hbm.at[idx_v.at[pl.ds(c * CH, CH)]], bufs[k], gsem)

    def st(c, k):
        return pltpu.async_copy(
            bufs[k], out_hbm.at[pl.ds(base + c * CH, CH)], ssem)

    _pipelined(NCH1, ld, st)


_gather = pl.kernel(
    _gather_body,
    out_type=jax.ShapeDtypeStruct((NPAD, D_IN), jnp.float32),
    scratch_types=[
        pltpu.VMEM((RPW,), jnp.int32),
        pltpu.VMEM((CH, D_IN), jnp.float32),
        pltpu.VMEM((CH, D_IN), jnp.float32),
        pltpu.SemaphoreType.DMA,
        pltpu.SemaphoreType.DMA,
    ],
    **_MESH,
)


def _combine_body(ys_hbm, exp_hbm, pos_hbm, tok_hbm, out1_hbm, out2_hbm,
                  idx_v, buf0, buf1, gsem, ssem):
    wid = lax.axis_index("s") * NC + lax.axis_index("c")
    base = wid * TPW
    bufs = (buf0, buf1)

    pltpu.sync_copy(pos_hbm.at[pl.ds(base, TPW)], idx_v)
    _pipelined(
        NCH3,
        lambda c, k: pltpu.async_copy(
            ys_hbm.at[idx_v.at[pl.ds(c * CH, CH)]], bufs[k], gsem),
        lambda c, k: pltpu.async_copy(
            bufs[k], out1_hbm.at[pl.ds(base + c * CH, CH)], ssem),
    )

    pltpu.sync_copy(tok_hbm.at[pl.ds(base, TPW)], idx_v)
    _pipelined(
        NCH3,
        lambda c, k: pltpu.async_copy(
            exp_hbm.at[idx_v.at[pl.ds(c * CH, CH)]], bufs[k], gsem),
        lambda c, k: pltpu.async_copy(
            bufs[k], out2_hbm.at[pl.ds(base + c * CH, CH)], ssem),
    )


_combine = pl.kernel(
    _combine_body,
    out_type=[
        jax.ShapeDtypeStruct((B, D_OUT), jnp.float32),
        jax.ShapeDtypeStruct((B, D_OUT), jnp.float32),
    ],
    scratch_types=[
        pltpu.VMEM((TPW,), jnp.int32),
        pltpu.VMEM((CH, D_OUT), jnp.float32),
        pltpu.VMEM((CH, D_OUT), jnp.float32),
        pltpu.SemaphoreType.DMA,
        pltpu.SemaphoreType.DMA,
    ],
    **_MESH,
)


def _mm_body(bexp_ref, xs_ref, w_ref, b_ref, ls_ref, ys_ref, exp_ref):
    y = lax.dot_general(
        xs_ref[...], w_ref[0], (((1,), (1,)), ((), ())),
        preferred_element_type=jnp.float32,
    )
    ys_ref[...] = y + b_ref[0]
    exp_ref[...] = jnp.exp(ls_ref[...])


_mm_grid = pltpu.PrefetchScalarGridSpec(
    num_scalar_prefetch=1,
    grid=(NB,),
    in_specs=[
        pl.BlockSpec((BM, D_IN), lambda j, be: (j, 0)),
        pl.BlockSpec((1, D_OUT, D_IN), lambda j, be: (be[j], 0, 0)),
        pl.BlockSpec((1, 1, D_OUT), lambda j, be: (be[j], 0, 0)),
        pl.BlockSpec((1, 1, D_OUT), lambda j, be: (be[j], 0, 0)),
    ],
    out_specs=[
        pl.BlockSpec((BM, D_OUT), lambda j, be: (j, 0)),
        pl.BlockSpec((1, 1, D_OUT), lambda j, be: (be[j], 0, 0)),
    ],
)

_mm = pl.pallas_call(
    _mm_body,
    grid_spec=_mm_grid,
    out_shape=[
        jax.ShapeDtypeStruct((NPAD, D_OUT), jnp.float32),
        jax.ShapeDtypeStruct((E, 1, D_OUT), jnp.float32),
    ],
)


def kernel(x, index, W, b, logstd):
    idx = index.astype(jnp.int32)

    oh = (idx[:, None] == jnp.arange(E, dtype=jnp.int32)[None, :]).astype(jnp.int32)
    counts = jnp.sum(oh, axis=0)
    nblk = (counts + BM - 1) // BM
    bends = jnp.cumsum(nblk)
    astart = ((bends - nblk) * BM).astype(jnp.int32)
    rank = jnp.cumsum(oh, axis=0) - 1
    myrank = jnp.take_along_axis(rank, idx[:, None], axis=1)[:, 0]
    pos = (astart[idx] + myrank).astype(jnp.int32)
    src = jnp.zeros((NPAD,), jnp.int32).at[pos].set(jnp.arange(B, dtype=jnp.int32))
    bexp = jnp.minimum(
        jnp.searchsorted(bends, jnp.arange(NB, dtype=jnp.int32), side="right"), E - 1
    ).astype(jnp.int32)

    xs = _gather(x, src)
    ys, exp3 = _mm(bexp, xs, W, b[:, None, :], logstd[:, None, :])
    out1, out2 = _combine(ys, exp3.reshape(E, D_OUT), pos, idx)
    return (out1, out2)


# speedup vs baseline: 1.7090x; 1.7090x over previous
import jax
import jax.numpy as jnp
from jax import lax
from jax.experimental import pallas as pl
from jax.experimental.pallas import tpu as pltpu
from jax.experimental.pallas import tpu_sc as plsc

E = 8
B = 8192
D_IN = 2048
D_OUT = 2048

BM = 256
NB = B // BM + E
NPAD = NB * BM

try:
    _info = plsc.get_sparse_core_info()
    NC, NS = int(_info.num_cores), int(_info.num_subcores)
except Exception:
    NC, NS = 2, 16
NW = NC * NS
RPW = NPAD // NW
TPW = B // NW
CH = 16
NCH1 = RPW // CH
NCH3 = TPW // CH

REP = 128

_MESH = dict(mesh=plsc.VectorSubcoreMesh(core_axis_name="c", subcore_axis_name="s"))


def _pipelined(nch, start_load, start_store):
    lh, sh = {}, {}
    lh[0] = start_load(0, 0)
    for c in range(nch):
        k = c % 2
        lh[c].wait()
        if c + 1 < nch:
            if c - 1 >= 0:
                sh[c - 1].wait()
            lh[c + 1] = start_load(c + 1, 1 - k)
        sh[c] = start_store(c, k)
    sh[nch - 1].wait()
    if nch >= 2:
        sh[nch - 2].wait()


def _gather_body(x_hbm, src_hbm, out_hbm, idx_v, buf0, buf1, gsem, ssem):
    wid = lax.axis_index("s") * NC + lax.axis_index("c")
    base = wid * RPW
    bufs = (buf0, buf1)
    pltpu.sync_copy(src_hbm.at[pl.ds(base, RPW)], idx_v)

    def ld(c, k):
        return pltpu.async_copy(
            x_hbm.at[idx_v.at[pl.ds(c * CH, CH)]], bufs[k], gsem)

    def st(c, k):
        return pltpu.async_copy(
            bufs[k], out_hbm.at[pl.ds(base + c * CH, CH)], ssem)

    _pipelined(NCH1, ld, st)


_gather = pl.kernel(
    _gather_body,
    out_type=jax.ShapeDtypeStruct((NPAD, D_IN), jnp.float32),
    scratch_types=[
        pltpu.VMEM((RPW,), jnp.int32),
        pltpu.VMEM((CH, D_IN), jnp.float32),
        pltpu.VMEM((CH, D_IN), jnp.float32),
        pltpu.SemaphoreType.DMA,
        pltpu.SemaphoreType.DMA,
    ],
    **_MESH,
)


def _combine_body(ys_hbm, exp_hbm, pos_hbm, tok_hbm, out1_hbm, out2_hbm,
                  idx_v, buf0, buf1, gsem, ssem):
    wid = lax.axis_index("s") * NC + lax.axis_index("c")
    base = wid * TPW
    bufs = (buf0, buf1)

    pltpu.sync_copy(pos_hbm.at[pl.ds(base, TPW)], idx_v)
    _pipelined(
        NCH3,
        lambda c, k: pltpu.async_copy(
            ys_hbm.at[idx_v.at[pl.ds(c * CH, CH)]], bufs[k], gsem),
        lambda c, k: pltpu.async_copy(
            bufs[k], out1_hbm.at[pl.ds(base + c * CH, CH)], ssem),
    )

    pltpu.sync_copy(tok_hbm.at[pl.ds(base, TPW)], idx_v)
    _pipelined(
        NCH3,
        lambda c, k: pltpu.async_copy(
            exp_hbm.at[idx_v.at[pl.ds(c * CH, CH)]], bufs[k], gsem),
        lambda c, k: pltpu.async_copy(
            bufs[k], out2_hbm.at[pl.ds(base + c * CH, CH)], ssem),
    )


_combine = pl.kernel(
    _combine_body,
    out_type=[
        jax.ShapeDtypeStruct((B, D_OUT), jnp.float32),
        jax.ShapeDtypeStruct((B, D_OUT), jnp.float32),
    ],
    scratch_types=[
        pltpu.VMEM((TPW,), jnp.int32),
        pltpu.VMEM((CH, D_OUT), jnp.float32),
        pltpu.VMEM((CH, D_OUT), jnp.float32),
        pltpu.SemaphoreType.DMA,
        pltpu.SemaphoreType.DMA,
    ],
    **_MESH,
)


def _mm_body(bexp_ref, xs_ref, w_ref, b_ref, ys_ref):
    y = lax.dot_general(
        xs_ref[...], w_ref[0], (((1,), (1,)), ((), ())),
        preferred_element_type=jnp.float32,
    )
    ys_ref[...] = y + b_ref[0]


_mm_grid = pltpu.PrefetchScalarGridSpec(
    num_scalar_prefetch=1,
    grid=(NB,),
    in_specs=[
        pl.BlockSpec((BM, D_IN), lambda j, be: (j, 0)),
        pl.BlockSpec((1, D_OUT, D_IN), lambda j, be: (be[j], 0, 0)),
        pl.BlockSpec((1, 1, D_OUT), lambda j, be: (be[j], 0, 0)),
    ],
    out_specs=pl.BlockSpec((BM, D_OUT), lambda j, be: (j, 0)),
)

_mm = pl.pallas_call(
    _mm_body,
    grid_spec=_mm_grid,
    out_shape=jax.ShapeDtypeStruct((NPAD, D_OUT), jnp.float32),
)


def _expand_body(ls_ref, rep_ref):
    rep_ref[...] = jnp.broadcast_to(jnp.exp(ls_ref[0]), (REP, D_OUT))


_expand = pl.pallas_call(
    _expand_body,
    grid=(E,),
    in_specs=[pl.BlockSpec((1, 1, D_OUT), lambda e: (e, 0, 0))],
    out_specs=pl.BlockSpec((REP, D_OUT), lambda e: (e, 0)),
    out_shape=jax.ShapeDtypeStruct((E * REP, D_OUT), jnp.float32),
)


def kernel(x, index, W, b, logstd):
    idx = index.astype(jnp.int32)

    oh = (idx[:, None] == jnp.arange(E, dtype=jnp.int32)[None, :]).astype(jnp.int32)
    counts = jnp.sum(oh, axis=0)
    nblk = (counts + BM - 1) // BM
    bends = jnp.cumsum(nblk)
    astart = ((bends - nblk) * BM).astype(jnp.int32)
    rank = jnp.cumsum(oh, axis=0) - 1
    myrank = jnp.take_along_axis(rank, idx[:, None], axis=1)[:, 0]
    pos = (astart[idx] + myrank).astype(jnp.int32)
    arp = jnp.arange(NPAD, dtype=jnp.int32)
    src = (arp % B).at[pos].set(jnp.arange(B, dtype=jnp.int32))
    bexp = jnp.minimum(
        jnp.searchsorted(bends, jnp.arange(NB, dtype=jnp.int32), side="right"), E - 1
    ).astype(jnp.int32)
    arb = jnp.arange(B, dtype=jnp.int32)
    idx2 = idx * REP + (arb % REP)

    xs = _gather(x, src)
    ys = _mm(bexp, xs, W, b[:, None, :])
    rep = _expand(logstd[:, None, :])
    out1, out2 = _combine(ys, rep, pos, idx2)
    return (out1, out2)


# speedup vs baseline: 1.8689x; 1.0935x over previous
import jax
import jax.numpy as jnp
from jax import lax
from jax.experimental import pallas as pl
from jax.experimental.pallas import tpu as pltpu
from jax.experimental.pallas import tpu_sc as plsc

E = 8
B = 8192
D_IN = 2048
D_OUT = 2048

BM = 256
NB = B // BM + E
NPAD = NB * BM

try:
    _info = plsc.get_sparse_core_info()
    NC, NS = int(_info.num_cores), int(_info.num_subcores)
except Exception:
    NC, NS = 2, 16
NW = NC * NS
RPW = NPAD // NW
TPW = B // NW
CH = 16
NCH1 = RPW // CH
NCH3 = TPW // CH

REP = 128

_MESH = dict(mesh=plsc.VectorSubcoreMesh(core_axis_name="c", subcore_axis_name="s"))


def _pipelined(nch, start_load, start_store):
    lh, sh = {}, {}
    lh[0] = start_load(0, 0)
    for c in range(nch):
        k = c % 2
        lh[c].wait()
        if c + 1 < nch:
            if c - 1 >= 0:
                sh[c - 1].wait()
            lh[c + 1] = start_load(c + 1, 1 - k)
        sh[c] = start_store(c, k)
    sh[nch - 1].wait()
    if nch >= 2:
        sh[nch - 2].wait()


def _gather_body(x_hbm, src_hbm, out_hbm, idx_v, buf0, buf1, gsem, ssem):
    wid = lax.axis_index("s") * NC + lax.axis_index("c")
    base = wid * RPW
    bufs = (buf0, buf1)
    pltpu.sync_copy(src_hbm.at[pl.ds(base, RPW)], idx_v)

    def ld(c, k):
        return pltpu.async_copy(
            x_hbm.at[idx_v.at[pl.ds(c * CH, CH)]], bufs[k], gsem)

    def st(c, k):
        return pltpu.async_copy(
            bufs[k], out_hbm.at[pl.ds(base + c * CH, CH)], ssem)

    _pipelined(NCH1, ld, st)


_gather = pl.kernel(
    _gather_body,
    out_type=jax.ShapeDtypeStruct((NPAD, D_IN), jnp.float32),
    scratch_types=[
        pltpu.VMEM((RPW,), jnp.int32),
        pltpu.VMEM((CH, D_IN), jnp.float32),
        pltpu.VMEM((CH, D_IN), jnp.float32),
        pltpu.SemaphoreType.DMA,
        pltpu.SemaphoreType.DMA,
    ],
    **_MESH,
)


def _combine_body(ys_hbm, exp_hbm, pos_hbm, tok_hbm, out1_hbm, out2_hbm,
                  idx_v, buf0, buf1, gsem, ssem):
    wid = lax.axis_index("s") * NC + lax.axis_index("c")
    base = wid * TPW
    bufs = (buf0, buf1)

    pltpu.sync_copy(pos_hbm.at[pl.ds(base, TPW)], idx_v)
    _pipelined(
        NCH3,
        lambda c, k: pltpu.async_copy(
            ys_hbm.at[idx_v.at[pl.ds(c * CH, CH)]], bufs[k], gsem),
        lambda c, k: pltpu.async_copy(
            bufs[k], out1_hbm.at[pl.ds(base + c * CH, CH)], ssem),
    )

    pltpu.sync_copy(tok_hbm.at[pl.ds(base, TPW)], idx_v)
    _pipelined(
        NCH3,
        lambda c, k: pltpu.async_copy(
            exp_hbm.at[idx_v.at[pl.ds(c * CH, CH)]], bufs[k], gsem),
        lambda c, k: pltpu.async_copy(
            bufs[k], out2_hbm.at[pl.ds(base + c * CH, CH)], ssem),
    )


_combine = pl.kernel(
    _combine_body,
    out_type=[
        jax.ShapeDtypeStruct((B, D_OUT), jnp.float32),
        jax.ShapeDtypeStruct((B, D_OUT), jnp.float32),
    ],
    scratch_types=[
        pltpu.VMEM((TPW,), jnp.int32),
        pltpu.VMEM((CH, D_OUT), jnp.float32),
        pltpu.VMEM((CH, D_OUT), jnp.float32),
        pltpu.SemaphoreType.DMA,
        pltpu.SemaphoreType.DMA,
    ],
    **_MESH,
)


def _mm_body(bexp_ref, xs_ref, w_ref, b_ref, ys_ref):
    y = lax.dot_general(
        xs_ref[...], w_ref[0], (((1,), (1,)), ((), ())),
        preferred_element_type=jnp.float32,
    )
    ys_ref[...] = y + b_ref[0]


_mm_grid = pltpu.PrefetchScalarGridSpec(
    num_scalar_prefetch=1,
    grid=(NB,),
    in_specs=[
        pl.BlockSpec((BM, D_IN), lambda j, be: (j, 0)),
        pl.BlockSpec((1, D_OUT, D_IN), lambda j, be: (be[j], 0, 0)),
        pl.BlockSpec((1, 1, D_OUT), lambda j, be: (be[j], 0, 0)),
    ],
    out_specs=pl.BlockSpec((BM, D_OUT), lambda j, be: (j, 0)),
)

_mm = pl.pallas_call(
    _mm_body,
    grid_spec=_mm_grid,
    out_shape=jax.ShapeDtypeStruct((NPAD, D_OUT), jnp.float32),
)


def _expand_body(ls_ref, rep_ref):
    rep_ref[...] = jnp.broadcast_to(jnp.exp(ls_ref[0]), (REP, D_OUT))


_expand = pl.pallas_call(
    _expand_body,
    grid=(E,),
    in_specs=[pl.BlockSpec((1, 1, D_OUT), lambda e: (e, 0, 0))],
    out_specs=pl.BlockSpec((REP, D_OUT), lambda e: (e, 0)),
    out_shape=jax.ShapeDtypeStruct((E * REP, D_OUT), jnp.float32),
)


def kernel(x, index, W, b, logstd):
    idx = index.astype(jnp.int32)

    oh = (jnp.arange(E, dtype=jnp.int32)[:, None] == idx[None, :]).astype(jnp.int32)
    csum = jnp.cumsum(oh, axis=1)
    counts = csum[:, -1]
    nblk = (counts + BM - 1) // BM
    bends = jnp.cumsum(nblk)
    astart = ((bends - nblk) * BM).astype(jnp.int32)
    pos = jnp.sum(oh * (astart[:, None] + csum - 1), axis=0).astype(jnp.int32)
    arb = jnp.arange(B, dtype=jnp.int32)
    arp = jnp.arange(NPAD, dtype=jnp.int32)
    inv = jnp.zeros((NPAD,), jnp.int32).at[pos].add(arb + 1)
    src = jnp.where(inv == 0, arp % B, inv - 1)
    bexp = jnp.minimum(
        jnp.sum((jnp.arange(NB, dtype=jnp.int32)[None, :] >= bends[:, None])
                .astype(jnp.int32), axis=0), E - 1).astype(jnp.int32)
    idx2 = idx * REP + (arb % REP)

    xs = _gather(x, src)
    ys = _mm(bexp, xs, W, b[:, None, :])
    rep = _expand(logstd[:, None, :])
    out1, out2 = _combine(ys, rep, pos, idx2)
    return (out1, out2)


# speedup vs baseline: 2.0242x; 1.0831x over previous
import jax
import jax.numpy as jnp
from jax import lax
from jax.experimental import pallas as pl
from jax.experimental.pallas import tpu as pltpu
from jax.experimental.pallas import tpu_sc as plsc

E = 8
B = 8192
D_IN = 2048
D_OUT = 2048

BM = 256
NB = B // BM + E
NPAD = NB * BM

try:
    _info = plsc.get_sparse_core_info()
    NC, NS = int(_info.num_cores), int(_info.num_subcores)
except Exception:
    NC, NS = 2, 16
NW = NC * NS
RPW = NPAD // NW
TPW = B // NW
CH = 16
NCH1 = RPW // CH
NCH3 = TPW // CH

_MESH = dict(mesh=plsc.VectorSubcoreMesh(core_axis_name="c", subcore_axis_name="s"))


def _pipelined(nch, start_load, start_store):
    lh, sh = {}, {}
    lh[0] = start_load(0, 0)
    for c in range(nch):
        k = c % 2
        lh[c].wait()
        if c + 1 < nch:
            if c - 1 >= 0:
                sh[c - 1].wait()
            lh[c + 1] = start_load(c + 1, 1 - k)
        sh[c] = start_store(c, k)
    sh[nch - 1].wait()
    if nch >= 2:
        sh[nch - 2].wait()


def _gather_body(x_hbm, src_hbm, out_hbm, idx_v, buf0, buf1, gsem, ssem):
    wid = lax.axis_index("s") * NC + lax.axis_index("c")
    base = wid * RPW
    bufs = (buf0, buf1)
    pltpu.sync_copy(src_hbm.at[pl.ds(base, RPW)], idx_v)

    def ld(c, k):
        return pltpu.async_copy(
            x_hbm.at[idx_v.at[pl.ds(c * CH, CH)]], bufs[k], gsem)

    def st(c, k):
        return pltpu.async_copy(
            bufs[k], out_hbm.at[pl.ds(base + c * CH, CH)], ssem)

    _pipelined(NCH1, ld, st)


_gather = pl.kernel(
    _gather_body,
    out_type=jax.ShapeDtypeStruct((NPAD, D_IN), jnp.float32),
    scratch_types=[
        pltpu.VMEM((RPW,), jnp.int32),
        pltpu.VMEM((CH, D_IN), jnp.float32),
        pltpu.VMEM((CH, D_IN), jnp.float32),
        pltpu.SemaphoreType.DMA,
        pltpu.SemaphoreType.DMA,
    ],
    **_MESH,
)


def _combine_body(ys_hbm, pos_hbm, out1_hbm, idx_v, buf0, buf1, gsem, ssem):
    wid = lax.axis_index("s") * NC + lax.axis_index("c")
    base = wid * TPW
    bufs = (buf0, buf1)

    pltpu.sync_copy(pos_hbm.at[pl.ds(base, TPW)], idx_v)
    _pipelined(
        NCH3,
        lambda c, k: pltpu.async_copy(
            ys_hbm.at[idx_v.at[pl.ds(c * CH, CH)]], bufs[k], gsem),
        lambda c, k: pltpu.async_copy(
            bufs[k], out1_hbm.at[pl.ds(base + c * CH, CH)], ssem),
    )


_combine = pl.kernel(
    _combine_body,
    out_type=jax.ShapeDtypeStruct((B, D_OUT), jnp.float32),
    scratch_types=[
        pltpu.VMEM((TPW,), jnp.int32),
        pltpu.VMEM((CH, D_OUT), jnp.float32),
        pltpu.VMEM((CH, D_OUT), jnp.float32),
        pltpu.SemaphoreType.DMA,
        pltpu.SemaphoreType.DMA,
    ],
    **_MESH,
)


def _mm_body(bexp_ref, xs_ref, w_ref, b_ref, ys_ref):
    y = lax.dot_general(
        xs_ref[...], w_ref[0], (((1,), (1,)), ((), ())),
        preferred_element_type=jnp.float32,
    )
    ys_ref[...] = y + b_ref[0]


_mm_grid = pltpu.PrefetchScalarGridSpec(
    num_scalar_prefetch=1,
    grid=(NB,),
    in_specs=[
        pl.BlockSpec((BM, D_IN), lambda j, be: (j, 0)),
        pl.BlockSpec((1, D_OUT, D_IN), lambda j, be: (be[j], 0, 0)),
        pl.BlockSpec((1, 1, D_OUT), lambda j, be: (be[j], 0, 0)),
    ],
    out_specs=pl.BlockSpec((BM, D_OUT), lambda j, be: (j, 0)),
)

_mm = pl.pallas_call(
    _mm_body,
    grid_spec=_mm_grid,
    out_shape=jax.ShapeDtypeStruct((NPAD, D_OUT), jnp.float32),
)


def _std_body(idxc_ref, ls_ref, out_ref):
    ic = idxc_ref[0]
    acc = jnp.zeros((BM, D_OUT), jnp.float32)
    for e in range(E):
        row = jnp.exp(ls_ref[e, 0, :])[None, :]
        acc = jnp.where(ic == e, row, acc)
    out_ref[...] = acc


_std = pl.pallas_call(
    _std_body,
    grid=(B // BM,),
    in_specs=[pl.BlockSpec((1, BM, 1), lambda j: (j, 0, 0)),
              pl.BlockSpec((E, 1, D_OUT), lambda j: (0, 0, 0))],
    out_specs=pl.BlockSpec((BM, D_OUT), lambda j: (j, 0)),
    out_shape=jax.ShapeDtypeStruct((B, D_OUT), jnp.float32),
)


def kernel(x, index, W, b, logstd):
    idx = index.astype(jnp.int32)

    oh = (jnp.arange(E, dtype=jnp.int32)[:, None] == idx[None, :]).astype(jnp.int32)
    csum = jnp.cumsum(oh, axis=1)
    counts = csum[:, -1]
    nblk = (counts + BM - 1) // BM
    bends = jnp.cumsum(nblk)
    astart = ((bends - nblk) * BM).astype(jnp.int32)
    pos = jnp.sum(oh * (astart[:, None] + csum - 1), axis=0).astype(jnp.int32)
    arb = jnp.arange(B, dtype=jnp.int32)
    arp = jnp.arange(NPAD, dtype=jnp.int32)
    inv = jnp.zeros((NPAD,), jnp.int32).at[pos].add(arb + 1)
    src = jnp.where(inv == 0, arp % B, inv - 1)
    bexp = jnp.minimum(
        jnp.sum((jnp.arange(NB, dtype=jnp.int32)[None, :] >= bends[:, None])
                .astype(jnp.int32), axis=0), E - 1).astype(jnp.int32)

    xs = _gather(x, src)
    ys = _mm(bexp, xs, W, b[:, None, :])
    out2 = _std(idx.reshape(B // BM, BM, 1), logstd[:, None, :])
    out1 = _combine(ys, pos)
    return (out1, out2)


# speedup vs baseline: 2.1874x; 1.0807x over previous
import jax
import jax.numpy as jnp
from jax import lax
from jax.experimental import pallas as pl
from jax.experimental.pallas import tpu as pltpu
from jax.experimental.pallas import tpu_sc as plsc

E = 8
B = 8192
D_IN = 2048
D_OUT = 2048

BM = 256
NB = B // BM + E
NPAD = NB * BM

try:
    _info = plsc.get_sparse_core_info()
    NC, NS = int(_info.num_cores), int(_info.num_subcores)
except Exception:
    NC, NS = 2, 16
NW = NC * NS
RPW = NPAD // NW
TPW = B // NW
CH = 16
NCH1 = RPW // CH
NCH3 = TPW // CH

_MESH = dict(mesh=plsc.VectorSubcoreMesh(core_axis_name="c", subcore_axis_name="s"))


def _pipelined(nch, start_load, start_store):
    lh, sh = {}, {}
    lh[0] = start_load(0, 0)
    for c in range(nch):
        k = c % 2
        lh[c].wait()
        if c + 1 < nch:
            if c - 1 >= 0:
                sh[c - 1].wait()
            lh[c + 1] = start_load(c + 1, 1 - k)
        sh[c] = start_store(c, k)
    sh[nch - 1].wait()
    if nch >= 2:
        sh[nch - 2].wait()


def _disp_body(x_hbm, pos3_hbm, xs_hbm, idx2_v, buf0, buf1, gsem, ssem):
    wid = lax.axis_index("s") * NC + lax.axis_index("c")
    base = wid * TPW
    bufs = (buf0, buf1)
    pltpu.sync_copy(pos3_hbm.at[wid], idx2_v)

    def ld(c, k):
        return pltpu.async_copy(
            x_hbm.at[pl.ds(base + c * CH, CH)], bufs[k], gsem)

    def st(c, k):
        return pltpu.async_copy(bufs[k], xs_hbm.at[idx2_v.at[c]], ssem)

    _pipelined(NCH3, ld, st)


_disp = pl.kernel(
    _disp_body,
    out_type=jax.ShapeDtypeStruct((NPAD, D_IN), jnp.float32),
    scratch_types=[
        pltpu.VMEM((NCH3, CH), jnp.int32),
        pltpu.VMEM((CH, D_IN), jnp.float32),
        pltpu.VMEM((CH, D_IN), jnp.float32),
        pltpu.SemaphoreType.DMA,
        pltpu.SemaphoreType.DMA,
    ],
    **_MESH,
)


def _combine_body(ys_hbm, pos_hbm, out1_hbm, idx_v, buf0, buf1, gsem, ssem):
    wid = lax.axis_index("s") * NC + lax.axis_index("c")
    base = wid * TPW
    bufs = (buf0, buf1)

    pltpu.sync_copy(pos_hbm.at[pl.ds(base, TPW)], idx_v)
    _pipelined(
        NCH3,
        lambda c, k: pltpu.async_copy(
            ys_hbm.at[idx_v.at[pl.ds(c * CH, CH)]], bufs[k], gsem),
        lambda c, k: pltpu.async_copy(
            bufs[k], out1_hbm.at[pl.ds(base + c * CH, CH)], ssem),
    )


_combine = pl.kernel(
    _combine_body,
    out_type=jax.ShapeDtypeStruct((B, D_OUT), jnp.float32),
    scratch_types=[
        pltpu.VMEM((TPW,), jnp.int32),
        pltpu.VMEM((CH, D_OUT), jnp.float32),
        pltpu.VMEM((CH, D_OUT), jnp.float32),
        pltpu.SemaphoreType.DMA,
        pltpu.SemaphoreType.DMA,
    ],
    **_MESH,
)


def _mm_body(scal_ref, xs_ref, w_ref, b_ref, ys_ref):
    j = pl.program_id(0)

    @pl.when(j < scal_ref[NB])
    def _():
        y = lax.dot_general(
            xs_ref[...], w_ref[0], (((1,), (1,)), ((), ())),
            preferred_element_type=jnp.float32,
        )
        ys_ref[...] = y + b_ref[0]


_mm_grid = pltpu.PrefetchScalarGridSpec(
    num_scalar_prefetch=1,
    grid=(NB,),
    in_specs=[
        pl.BlockSpec((BM, D_IN), lambda j, be: (j, 0)),
        pl.BlockSpec((1, D_OUT, D_IN), lambda j, be: (be[j], 0, 0)),
        pl.BlockSpec((1, 1, D_OUT), lambda j, be: (be[j], 0, 0)),
    ],
    out_specs=pl.BlockSpec((BM, D_OUT), lambda j, be: (j, 0)),
)

_mm = pl.pallas_call(
    _mm_body,
    grid_spec=_mm_grid,
    out_shape=jax.ShapeDtypeStruct((NPAD, D_OUT), jnp.float32),
)


def _std_body(idxc_ref, ls_ref, out_ref):
    ic = idxc_ref[0]
    acc = jnp.zeros((BM, D_OUT), jnp.float32)
    for e in range(E):
        row = jnp.exp(ls_ref[e, 0, :])[None, :]
        acc = jnp.where(ic == e, row, acc)
    out_ref[...] = acc


_std = pl.pallas_call(
    _std_body,
    grid=(B // BM,),
    in_specs=[pl.BlockSpec((1, BM, 1), lambda j: (j, 0, 0)),
              pl.BlockSpec((E, 1, D_OUT), lambda j: (0, 0, 0))],
    out_specs=pl.BlockSpec((BM, D_OUT), lambda j: (j, 0)),
    out_shape=jax.ShapeDtypeStruct((B, D_OUT), jnp.float32),
)


def kernel(x, index, W, b, logstd):
    idx = index.astype(jnp.int32)

    oh = (jnp.arange(E, dtype=jnp.int32)[:, None] == idx[None, :]).astype(jnp.int32)
    csum = jnp.cumsum(oh, axis=1)
    counts = csum[:, -1]
    nblk = (counts + BM - 1) // BM
    bends = jnp.cumsum(nblk)
    astart = ((bends - nblk) * BM).astype(jnp.int32)
    pos = jnp.sum(oh * (astart[:, None] + csum - 1), axis=0).astype(jnp.int32)
    bexp = jnp.minimum(
        jnp.sum((jnp.arange(NB, dtype=jnp.int32)[None, :] >= bends[:, None])
                .astype(jnp.int32), axis=0), E - 1).astype(jnp.int32)
    scal = jnp.concatenate([bexp, bends[-1:].astype(jnp.int32)])

    xs = _disp(x, pos.reshape(NW, NCH3, CH))
    ys = _mm(scal, xs, W, b[:, None, :])
    out2 = _std(idx.reshape(B // BM, BM, 1), logstd[:, None, :])
    out1 = _combine(ys, pos)
    return (out1, out2)


# speedup vs baseline: 2.1883x; 1.0004x over previous
import jax
import jax.numpy as jnp
from jax import lax
from jax.experimental import pallas as pl
from jax.experimental.pallas import tpu as pltpu
from jax.experimental.pallas import tpu_sc as plsc

E = 8
B = 8192
D_IN = 2048
D_OUT = 2048

BM = 256
NB = B // BM + E
NPAD = NB * BM

try:
    _info = plsc.get_sparse_core_info()
    NC, NS = int(_info.num_cores), int(_info.num_subcores)
except Exception:
    NC, NS = 2, 16
NW = NC * NS
TPW = B // NW
CH = 16
NCH3 = TPW // CH

_MESH = dict(mesh=plsc.VectorSubcoreMesh(core_axis_name="c", subcore_axis_name="s"))


def _pipelined(nch, start_load, start_store, nbuf=2):
    lh, sh = {}, {}
    for c in range(min(nbuf - 1, nch)):
        lh[c] = start_load(c, c % nbuf)
    for c in range(nch):
        k = c % nbuf
        lh[c].wait()
        n = c + nbuf - 1
        if n < nch:
            if c - 1 >= 0:
                sh[c - 1].wait()
            lh[n] = start_load(n, n % nbuf)
        sh[c] = start_store(c, k)
    for c in range(max(0, nch - nbuf), nch):
        sh[c].wait()


def _disp_body(x_hbm, pos3_hbm, xs_hbm, idx2_v, buf0, buf1, buf2,
               g0, g1, g2, s0, s1, s2):
    wid = lax.axis_index("s") * NC + lax.axis_index("c")
    base = wid * TPW
    bufs = (buf0, buf1, buf2)
    gsems = (g0, g1, g2)
    ssems = (s0, s1, s2)
    pltpu.sync_copy(pos3_hbm.at[wid], idx2_v)

    def ld(c, k):
        return pltpu.async_copy(
            x_hbm.at[pl.ds(base + c * CH, CH)], bufs[k], gsems[k])

    def st(c, k):
        return pltpu.async_copy(bufs[k], xs_hbm.at[idx2_v.at[c]], ssems[k])

    _pipelined(NCH3, ld, st)


_disp = pl.kernel(
    _disp_body,
    out_type=jax.ShapeDtypeStruct((NPAD, D_IN), jnp.float32),
    scratch_types=[
        pltpu.VMEM((NCH3, CH), jnp.int32),
        pltpu.VMEM((CH, D_IN), jnp.float32),
        pltpu.VMEM((CH, D_IN), jnp.float32),
        pltpu.VMEM((CH, D_IN), jnp.float32),
        pltpu.SemaphoreType.DMA,
        pltpu.SemaphoreType.DMA,
        pltpu.SemaphoreType.DMA,
        pltpu.SemaphoreType.DMA,
        pltpu.SemaphoreType.DMA,
        pltpu.SemaphoreType.DMA,
    ],
    **_MESH,
)


def _combine_body(ys_hbm, pos_hbm, out1_hbm, idx_v, buf0, buf1, buf2,
                  g0, g1, g2, s0, s1, s2):
    wid = lax.axis_index("s") * NC + lax.axis_index("c")
    base = wid * TPW
    bufs = (buf0, buf1, buf2)
    gsems = (g0, g1, g2)
    ssems = (s0, s1, s2)

    pltpu.sync_copy(pos_hbm.at[pl.ds(base, TPW)], idx_v)
    _pipelined(
        NCH3,
        lambda c, k: pltpu.async_copy(
            ys_hbm.at[idx_v.at[pl.ds(c * CH, CH)]], bufs[k], gsems[k]),
        lambda c, k: pltpu.async_copy(
            bufs[k], out1_hbm.at[pl.ds(base + c * CH, CH)], ssems[k]),
    )


_combine = pl.kernel(
    _combine_body,
    out_type=jax.ShapeDtypeStruct((B, D_OUT), jnp.float32),
    scratch_types=[
        pltpu.VMEM((TPW,), jnp.int32),
        pltpu.VMEM((CH, D_OUT), jnp.float32),
        pltpu.VMEM((CH, D_OUT), jnp.float32),
        pltpu.VMEM((CH, D_OUT), jnp.float32),
        pltpu.SemaphoreType.DMA,
        pltpu.SemaphoreType.DMA,
        pltpu.SemaphoreType.DMA,
        pltpu.SemaphoreType.DMA,
        pltpu.SemaphoreType.DMA,
        pltpu.SemaphoreType.DMA,
    ],
    **_MESH,
)


def _mm_body(scal_ref, xs_ref, w_ref, b_ref, ys_ref):
    j = pl.program_id(0)

    @pl.when(j < scal_ref[NB])
    def _():
        y = lax.dot_general(
            xs_ref[...], w_ref[0], (((1,), (1,)), ((), ())),
            preferred_element_type=jnp.float32,
        )
        ys_ref[...] = y + b_ref[0]


_mm_grid = pltpu.PrefetchScalarGridSpec(
    num_scalar_prefetch=1,
    grid=(NB,),
    in_specs=[
        pl.BlockSpec((BM, D_IN), lambda j, be: (j, 0)),
        pl.BlockSpec((1, D_OUT, D_IN), lambda j, be: (be[j], 0, 0)),
        pl.BlockSpec((1, 1, D_OUT), lambda j, be: (be[j], 0, 0)),
    ],
    out_specs=pl.BlockSpec((BM, D_OUT), lambda j, be: (j, 0)),
)

_mm = pl.pallas_call(
    _mm_body,
    grid_spec=_mm_grid,
    out_shape=jax.ShapeDtypeStruct((NPAD, D_OUT), jnp.float32),
)


def _std_body(idxc_ref, ls_ref, out_ref):
    ic = idxc_ref[0]
    acc = jnp.zeros((BM, D_OUT), jnp.float32)
    for e in range(E):
        row = jnp.exp(ls_ref[e, 0, :])[None, :]
        acc = jnp.where(ic == e, row, acc)
    out_ref[...] = acc


_std = pl.pallas_call(
    _std_body,
    grid=(B // BM,),
    in_specs=[pl.BlockSpec((1, BM, 1), lambda j: (j, 0, 0)),
              pl.BlockSpec((E, 1, D_OUT), lambda j: (0, 0, 0))],
    out_specs=pl.BlockSpec((BM, D_OUT), lambda j: (j, 0)),
    out_shape=jax.ShapeDtypeStruct((B, D_OUT), jnp.float32),
)


def kernel(x, index, W, b, logstd):
    idx = index.astype(jnp.int32)

    oh = (jnp.arange(E, dtype=jnp.int32)[:, None] == idx[None, :]).astype(jnp.int32)
    csum = jnp.cumsum(oh, axis=1)
    counts = csum[:, -1]
    nblk = (counts + BM - 1) // BM
    bends = jnp.cumsum(nblk)
    astart = ((bends - nblk) * BM).astype(jnp.int32)
    pos = jnp.sum(oh * (astart[:, None] + csum - 1), axis=0).astype(jnp.int32)
    bexp = jnp.minimum(
        jnp.sum((jnp.arange(NB, dtype=jnp.int32)[None, :] >= bends[:, None])
                .astype(jnp.int32), axis=0), E - 1).astype(jnp.int32)
    scal = jnp.concatenate([bexp, bends[-1:].astype(jnp.int32)])

    xs = _disp(x, pos.reshape(NW, NCH3, CH))
    ys = _mm(scal, xs, W, b[:, None, :])
    out2 = _std(idx.reshape(B // BM, BM, 1), logstd[:, None, :])
    out1 = _combine(ys, pos)
    return (out1, out2)


# speedup vs baseline: 2.2279x; 1.0181x over previous
import jax
import jax.numpy as jnp
from jax import lax
from jax.experimental import pallas as pl
from jax.experimental.pallas import tpu as pltpu
from jax.experimental.pallas import tpu_sc as plsc

E = 8
B = 8192
D_IN = 2048
D_OUT = 2048

BM = 256
NB = B // BM + E
NPAD = NB * BM

try:
    _info = plsc.get_sparse_core_info()
    NC, NS = int(_info.num_cores), int(_info.num_subcores)
except Exception:
    NC, NS = 2, 16
NW = NC * NS
TPW = B // NW
CH = 16
NCH3 = TPW // CH

_MESH = dict(mesh=plsc.VectorSubcoreMesh(core_axis_name="c", subcore_axis_name="s"))


def _pipelined(nch, start_load, start_store, nbuf=3):
    lh, sh = {}, {}
    for c in range(min(nbuf - 1, nch)):
        lh[c] = start_load(c, c % nbuf)
    for c in range(nch):
        k = c % nbuf
        lh[c].wait()
        n = c + nbuf - 1
        if n < nch:
            if c - 1 >= 0:
                sh[c - 1].wait()
            lh[n] = start_load(n, n % nbuf)
        sh[c] = start_store(c, k)
    for c in range(max(0, nch - nbuf), nch):
        sh[c].wait()


def _disp_body(x_hbm, pos3_hbm, xs_hbm, idx2_v, buf0, buf1, buf2,
               g0, g1, g2, s0, s1, s2):
    wid = lax.axis_index("s") * NC + lax.axis_index("c")
    base = wid * TPW
    bufs = (buf0, buf1, buf2)
    gsems = (g0, g1, g2)
    ssems = (s0, s1, s2)
    pltpu.sync_copy(pos3_hbm.at[wid], idx2_v)

    def ld(c, k):
        return pltpu.async_copy(
            x_hbm.at[pl.ds(base + c * CH, CH)], bufs[k], gsems[k])

    def st(c, k):
        return pltpu.async_copy(bufs[k], xs_hbm.at[idx2_v.at[c]], ssems[k])

    _pipelined(NCH3, ld, st)


_disp = pl.kernel(
    _disp_body,
    out_type=jax.ShapeDtypeStruct((NPAD, D_IN), jnp.float32),
    scratch_types=[
        pltpu.VMEM((NCH3, CH), jnp.int32),
        pltpu.VMEM((CH, D_IN), jnp.float32),
        pltpu.VMEM((CH, D_IN), jnp.float32),
        pltpu.VMEM((CH, D_IN), jnp.float32),
        pltpu.SemaphoreType.DMA,
        pltpu.SemaphoreType.DMA,
        pltpu.SemaphoreType.DMA,
        pltpu.SemaphoreType.DMA,
        pltpu.SemaphoreType.DMA,
        pltpu.SemaphoreType.DMA,
    ],
    **_MESH,
)


def _combine_body(ys_hbm, pos_hbm, out1_hbm, idx_v, buf0, buf1, buf2,
                  g0, g1, g2, s0, s1, s2):
    wid = lax.axis_index("s") * NC + lax.axis_index("c")
    base = wid * TPW
    bufs = (buf0, buf1, buf2)
    gsems = (g0, g1, g2)
    ssems = (s0, s1, s2)

    pltpu.sync_copy(pos_hbm.at[pl.ds(base, TPW)], idx_v)
    _pipelined(
        NCH3,
        lambda c, k: pltpu.async_copy(
            ys_hbm.at[idx_v.at[pl.ds(c * CH, CH)]], bufs[k], gsems[k]),
        lambda c, k: pltpu.async_copy(
            bufs[k], out1_hbm.at[pl.ds(base + c * CH, CH)], ssems[k]),
    )


_combine = pl.kernel(
    _combine_body,
    out_type=jax.ShapeDtypeStruct((B, D_OUT), jnp.float32),
    scratch_types=[
        pltpu.VMEM((TPW,), jnp.int32),
        pltpu.VMEM((CH, D_OUT), jnp.float32),
        pltpu.VMEM((CH, D_OUT), jnp.float32),
        pltpu.VMEM((CH, D_OUT), jnp.float32),
        pltpu.SemaphoreType.DMA,
        pltpu.SemaphoreType.DMA,
        pltpu.SemaphoreType.DMA,
        pltpu.SemaphoreType.DMA,
        pltpu.SemaphoreType.DMA,
        pltpu.SemaphoreType.DMA,
    ],
    **_MESH,
)


def _mm_body(scal_ref, xs_ref, w_ref, b_ref, ys_ref):
    j = pl.program_id(0)

    @pl.when(j < scal_ref[NB])
    def _():
        y = lax.dot_general(
            xs_ref[...], w_ref[0], (((1,), (1,)), ((), ())),
            preferred_element_type=jnp.float32,
        )
        ys_ref[...] = y + b_ref[0]


_mm_grid = pltpu.PrefetchScalarGridSpec(
    num_scalar_prefetch=1,
    grid=(NB,),
    in_specs=[
        pl.BlockSpec((BM, D_IN), lambda j, be: (j, 0)),
        pl.BlockSpec((1, D_OUT, D_IN), lambda j, be: (be[j], 0, 0)),
        pl.BlockSpec((1, 1, D_OUT), lambda j, be: (be[j], 0, 0)),
    ],
    out_specs=pl.BlockSpec((BM, D_OUT), lambda j, be: (j, 0)),
)

_mm = pl.pallas_call(
    _mm_body,
    grid_spec=_mm_grid,
    out_shape=jax.ShapeDtypeStruct((NPAD, D_OUT), jnp.float32),
)


def _std_body(idxc_ref, ls_ref, out_ref):
    ic = idxc_ref[0]
    acc = jnp.zeros((BM, D_OUT), jnp.float32)
    for e in range(E):
        row = jnp.exp(ls_ref[e, 0, :])[None, :]
        acc = jnp.where(ic == e, row, acc)
    out_ref[...] = acc


_std = pl.pallas_call(
    _std_body,
    grid=(B // BM,),
    in_specs=[pl.BlockSpec((1, BM, 1), lambda j: (j, 0, 0)),
              pl.BlockSpec((E, 1, D_OUT), lambda j: (0, 0, 0))],
    out_specs=pl.BlockSpec((BM, D_OUT), lambda j: (j, 0)),
    out_shape=jax.ShapeDtypeStruct((B, D_OUT), jnp.float32),
)


def kernel(x, index, W, b, logstd):
    idx = index.astype(jnp.int32)

    oh = (jnp.arange(E, dtype=jnp.int32)[:, None] == idx[None, :]).astype(jnp.int32)
    csum = jnp.cumsum(oh, axis=1)
    counts = csum[:, -1]
    nblk = (counts + BM - 1) // BM
    bends = jnp.cumsum(nblk)
    astart = ((bends - nblk) * BM).astype(jnp.int32)
    pos = jnp.sum(oh * (astart[:, None] + csum - 1), axis=0).astype(jnp.int32)
    bexp = jnp.minimum(
        jnp.sum((jnp.arange(NB, dtype=jnp.int32)[None, :] >= bends[:, None])
                .astype(jnp.int32), axis=0), E - 1).astype(jnp.int32)
    scal = jnp.concatenate([bexp, bends[-1:].astype(jnp.int32)])

    xs = _disp(x, pos.reshape(NW, NCH3, CH))
    ys = _mm(scal, xs, W, b[:, None, :])
    out2 = _std(idx.reshape(B // BM, BM, 1), logstd[:, None, :])
    out1 = _combine(ys, pos)
    return (out1, out2)
